# Initial kernel scaffold; baseline (speedup 1.0000x reference)
import jax
import jax.numpy as jnp
from jax.experimental import pallas as pl


def kernel(x, pos_table):
    raise NotImplementedError("write your pallas kernel here")


# speedup vs baseline: 11.5454x; 11.5454x over previous
import jax
import jax.numpy as jnp
from jax.experimental import pallas as pl


def _add_body(x_ref, pos_ref, out_ref):
    out_ref[...] = x_ref[...] + pos_ref[...][:, None, :]


def kernel(x, pos_table):
    S, B, E = x.shape
    BS = 8
    grid = (S // BS,)
    return pl.pallas_call(
        _add_body,
        grid=grid,
        in_specs=[
            pl.BlockSpec((BS, B, E), lambda i: (i, 0, 0)),
            pl.BlockSpec((BS, E), lambda i: (i, 0)),
        ],
        out_specs=pl.BlockSpec((BS, B, E), lambda i: (i, 0, 0)),
        out_shape=jax.ShapeDtypeStruct((S, B, E), x.dtype),
    )(x, pos_table)
